# baseline (speedup 1.0000x reference)
import functools

import jax
import jax.numpy as jnp
from jax import lax
from jax.experimental import pallas as pl
from jax.experimental.pallas import tpu as pltpu
from jax.experimental.pallas import tpu_sc as plsc

_EPS = 1e-06

_NC = 2
_NS = 16
_NW = _NC * _NS


def _precompute_body(n_ranges, n_slices, vs, emb_ref, scale_ref, w_ref,
                     ids_ref, out_ref, cnt_ref):
    x = emb_ref[...]
    var = jnp.mean(x * x, axis=-1, keepdims=True)
    y = x * lax.rsqrt(var + _EPS) * scale_ref[...]
    z = jnp.dot(y, w_ref[...], preferred_element_type=jnp.float32)
    var2 = jnp.mean(z * z, axis=-1, keepdims=True)
    out_ref[...] = z * lax.rsqrt(var2 + _EPS)
    ids = ids_ref[...]
    coli = lax.broadcasted_iota(jnp.int32, (1, 128), 1)
    acc = jnp.zeros((1, 128), jnp.int32)
    for r in range(n_ranges):
        row = ids[r]
        for s in range(n_slices):
            c = jnp.sum(((row >= s * vs) & (row < (s + 1) * vs))
                        .astype(jnp.int32))
            acc = acc + jnp.where(coli == r * n_slices + s, c, 0)
    cnt_ref[...] = acc


def _precompute_table(emb_table, scale2d, proj_w, ids2d, n_ranges, n_slices):
    v, _ = emb_table.shape
    f = proj_w.shape[1]
    body = functools.partial(
        _precompute_body, n_ranges, n_slices, v // n_slices
    )
    return pl.pallas_call(
        body,
        out_shape=[
            jax.ShapeDtypeStruct((v, f), jnp.float32),
            jax.ShapeDtypeStruct((1, 128), jnp.int32),
        ],
    )(emb_table, scale2d, proj_w, ids2d)


@functools.lru_cache(maxsize=None)
def _make_gather(b, v, d):
    b_per_w = b // _NW
    chunk = 24
    chunks = [chunk] * (b_per_w // chunk)
    if b_per_w % chunk:
        chunks.append(b_per_w % chunk)
    offs = [sum(chunks[:i]) for i in range(len(chunks))]
    nchunks = len(chunks)
    mesh = plsc.VectorSubcoreMesh(core_axis_name="c", subcore_axis_name="s")

    @functools.partial(
        pl.kernel,
        mesh=mesh,
        out_type=jax.ShapeDtypeStruct((b, d), jnp.float32),
        scratch_types=[
            pltpu.VMEM((b_per_w,), jnp.int32),
            pltpu.VMEM((chunk, d), jnp.float32),
            pltpu.VMEM((chunk, d), jnp.float32),
            pltpu.SemaphoreType.DMA,
            pltpu.SemaphoreType.DMA,
        ],
    )
    def gather_kernel(ids_hbm, tab_hbm, out_hbm, idx_v, rows0, rows1, s0, s1):
        wid = lax.axis_index("s") * _NC + lax.axis_index("c")
        base = wid * b_per_w
        pltpu.sync_copy(ids_hbm.at[pl.ds(base, b_per_w)], idx_v)
        bufs = (rows0, rows1)
        sems = (s0, s1)
        copies = [None] * nchunks

        def start(c):
            return pltpu.async_copy(
                tab_hbm.at[idx_v.at[pl.ds(offs[c], chunks[c])]],
                bufs[c % 2].at[pl.ds(0, chunks[c])],
                sems[c % 2],
            )

        copies[0] = start(0)
        for c in range(nchunks):
            if c + 1 < nchunks:
                copies[c + 1] = start(c + 1)
            copies[c].wait()
            pltpu.sync_copy(
                bufs[c % 2].at[pl.ds(0, chunks[c])],
                out_hbm.at[pl.ds(base + offs[c], chunks[c])],
            )

    return gather_kernel


@functools.lru_cache(maxsize=None)
def _make_broadcast_write(b, v, d):
    n_slices = 8
    n_ranges = _NW // n_slices
    vs = v // n_slices
    tr = b // n_ranges
    grp = tr // 16
    mesh = plsc.VectorSubcoreMesh(core_axis_name="c", subcore_axis_name="s")

    @functools.partial(
        pl.kernel,
        mesh=mesh,
        out_type=jax.ShapeDtypeStruct((b, d), jnp.float32),
        scratch_types=[
            pltpu.VMEM((tr,), jnp.int32),
            pltpu.VMEM((vs, d), jnp.float32),
            pltpu.VMEM((48,), jnp.int32),
            pltpu.SemaphoreType.DMA,
        ],
    )
    def bw_kernel(ids_hbm, tab_hbm, cnt_hbm, out_hbm, ids_v, tab_v, cnt_v,
                  sem):
        wid = lax.axis_index("s") * _NC + lax.axis_index("c")
        slice_id = wid % n_slices
        range_id = wid // n_slices
        v0 = slice_id * vs
        t0 = range_id * tr
        pltpu.sync_copy(tab_hbm.at[pl.ds(v0, vs)], tab_v)
        pltpu.sync_copy(ids_hbm.at[pl.ds(t0, tr)], ids_v)
        pltpu.sync_copy(cnt_hbm.at[pl.ds(0, 48)], cnt_v)
        n_issued = cnt_v[pl.ds(wid, 16)][0]

        def scan(g, z):
            ids16 = ids_v[pl.ds(g * 16, 16)]
            rows16 = ids16 - v0
            mask = (ids16 >= v0) & (ids16 < v0 + vs)
            enc16 = jnp.where(mask, rows16, -1)
            gbase = t0 + g * 16
            for l in range(16):
                e = enc16[l]

                @pl.when(e >= 0)
                def _():
                    pltpu.async_copy(
                        tab_v.at[pl.ds(e, 1)],
                        out_hbm.at[pl.ds(gbase + l, 1)],
                        sem,
                    )

            return z

        lax.fori_loop(0, grp, scan, jnp.int32(0))

        def drain(_, carry):
            pltpu.make_async_copy(
                out_hbm.at[pl.ds(t0, 1)], tab_v.at[pl.ds(0, 1)], sem
            ).wait()
            return carry

        lax.fori_loop(0, n_issued, drain, jnp.int32(0))

    return bw_kernel


def kernel(input_ids, emb_table, hard_norm_scale, proj_w):
    bsz, seq = input_ids.shape
    f = proj_w.shape[1]
    n_slices = 8
    n_ranges = _NW // n_slices
    ids = input_ids.reshape(-1).astype(jnp.int32)
    ptab, cnt = _precompute_table(
        emb_table, hard_norm_scale.reshape(1, -1), proj_w,
        ids.reshape(n_ranges, -1), n_ranges, n_slices,
    )
    out = _make_broadcast_write(bsz * seq, emb_table.shape[0], f)(
        ids, ptab, cnt.reshape(-1)
    )
    return out.reshape(bsz, seq, f)


# speedup vs baseline: 1.3772x; 1.3772x over previous
import functools

import jax
import jax.numpy as jnp
from jax import lax
from jax.experimental import pallas as pl
from jax.experimental.pallas import tpu as pltpu
from jax.experimental.pallas import tpu_sc as plsc

_EPS = 1e-06

_NC = 2
_NS = 16
_NW = _NC * _NS


def _precompute_body(n_ranges, n_slices, vs, emb_ref, scale_ref, w_ref,
                     ids_ref, out_ref, cnt_ref):
    x = emb_ref[...]
    var = jnp.mean(x * x, axis=-1, keepdims=True)
    y = x * lax.rsqrt(var + _EPS) * scale_ref[...]
    z = jnp.dot(y, w_ref[...], preferred_element_type=jnp.float32)
    var2 = jnp.mean(z * z, axis=-1, keepdims=True)
    out_ref[...] = z * lax.rsqrt(var2 + _EPS)
    ids = ids_ref[...]
    coli = lax.broadcasted_iota(jnp.int32, (1, 128), 1)
    acc = jnp.zeros((1, 128), jnp.int32)
    for r in range(n_ranges):
        row = ids[r]
        for s in range(n_slices):
            c = jnp.sum(((row >= s * vs) & (row < (s + 1) * vs))
                        .astype(jnp.int32))
            acc = acc + jnp.where(coli == r * n_slices + s, c, 0)
    cnt_ref[...] = acc


def _precompute_table(emb_table, scale2d, proj_w, ids2d, n_ranges, n_slices):
    v, _ = emb_table.shape
    f = proj_w.shape[1]
    body = functools.partial(
        _precompute_body, n_ranges, n_slices, v // n_slices
    )
    return pl.pallas_call(
        body,
        out_shape=[
            jax.ShapeDtypeStruct((v, f), jnp.float32),
            jax.ShapeDtypeStruct((1, 128), jnp.int32),
        ],
    )(emb_table, scale2d, proj_w, ids2d)


@functools.lru_cache(maxsize=None)
def _make_gather(b, v, d):
    b_per_w = b // _NW
    chunk = 24
    chunks = [chunk] * (b_per_w // chunk)
    if b_per_w % chunk:
        chunks.append(b_per_w % chunk)
    offs = [sum(chunks[:i]) for i in range(len(chunks))]
    nchunks = len(chunks)
    mesh = plsc.VectorSubcoreMesh(core_axis_name="c", subcore_axis_name="s")

    @functools.partial(
        pl.kernel,
        mesh=mesh,
        out_type=jax.ShapeDtypeStruct((b, d), jnp.float32),
        scratch_types=[
            pltpu.VMEM((b_per_w,), jnp.int32),
            pltpu.VMEM((chunk, d), jnp.float32),
            pltpu.VMEM((chunk, d), jnp.float32),
            pltpu.SemaphoreType.DMA,
            pltpu.SemaphoreType.DMA,
        ],
    )
    def gather_kernel(ids_hbm, tab_hbm, out_hbm, idx_v, rows0, rows1, s0, s1):
        wid = lax.axis_index("s") * _NC + lax.axis_index("c")
        base = wid * b_per_w
        pltpu.sync_copy(ids_hbm.at[pl.ds(base, b_per_w)], idx_v)
        bufs = (rows0, rows1)
        sems = (s0, s1)
        copies = [None] * nchunks

        def start(c):
            return pltpu.async_copy(
                tab_hbm.at[idx_v.at[pl.ds(offs[c], chunks[c])]],
                bufs[c % 2].at[pl.ds(0, chunks[c])],
                sems[c % 2],
            )

        copies[0] = start(0)
        for c in range(nchunks):
            if c + 1 < nchunks:
                copies[c + 1] = start(c + 1)
            copies[c].wait()
            pltpu.sync_copy(
                bufs[c % 2].at[pl.ds(0, chunks[c])],
                out_hbm.at[pl.ds(base + offs[c], chunks[c])],
            )

    return gather_kernel


@functools.lru_cache(maxsize=None)
def _make_broadcast_write(b, v, d):
    n_slices = 8
    n_ranges = _NW // n_slices
    vs = v // n_slices
    tr = b // n_ranges
    grp = tr // 16
    mesh = plsc.VectorSubcoreMesh(core_axis_name="c", subcore_axis_name="s")

    @functools.partial(
        pl.kernel,
        mesh=mesh,
        out_type=jax.ShapeDtypeStruct((b, d), jnp.float32),
        scratch_types=[
            pltpu.VMEM((tr,), jnp.int32),
            pltpu.VMEM((vs, d), jnp.float32),
            pltpu.VMEM((48,), jnp.int32),
            pltpu.SemaphoreType.DMA,
        ],
    )
    def bw_kernel(ids_hbm, tab_hbm, cnt_hbm, out_hbm, ids_v, tab_v, cnt_v,
                  sem):
        wid = lax.axis_index("s") * _NC + lax.axis_index("c")
        slice_id = wid % n_slices
        range_id = wid // n_slices
        v0 = slice_id * vs
        t0 = range_id * tr
        pltpu.sync_copy(tab_hbm.at[pl.ds(v0, vs)], tab_v)
        pltpu.sync_copy(ids_hbm.at[pl.ds(t0, tr)], ids_v)
        pltpu.sync_copy(cnt_hbm.at[pl.ds(0, 48)], cnt_v)
        n_issued = cnt_v[pl.ds(wid, 16)][0]

        def scan(g, z):
            ids16 = ids_v[pl.ds(g * 16, 16)]
            rows16 = ids16 - v0
            mask = (ids16 >= v0) & (ids16 < v0 + vs)
            enc16 = jnp.where(mask, rows16, -1)
            gbase = t0 + g * 16
            for q in range(0, 16, 4):
                e4 = [enc16[q + i] for i in range(4)]
                any_hit = e4[0] & e4[1] & e4[2] & e4[3]

                @pl.when(any_hit >= 0)
                def _(q=q, e4=e4):
                    for i in range(4):
                        e = e4[i]

                        @pl.when(e >= 0)
                        def _(e=e, l=q + i):
                            pltpu.async_copy(
                                tab_v.at[pl.ds(e, 1)],
                                out_hbm.at[pl.ds(gbase + l, 1)],
                                sem,
                            )

            return z

        lax.fori_loop(0, grp, scan, jnp.int32(0))

        def drain(_, carry):
            pltpu.make_async_copy(
                out_hbm.at[pl.ds(t0, 1)], tab_v.at[pl.ds(0, 1)], sem
            ).wait()
            return carry

        lax.fori_loop(0, n_issued, drain, jnp.int32(0))

    return bw_kernel


def kernel(input_ids, emb_table, hard_norm_scale, proj_w):
    bsz, seq = input_ids.shape
    f = proj_w.shape[1]
    n_slices = 8
    n_ranges = _NW // n_slices
    ids = input_ids.reshape(-1).astype(jnp.int32)
    ptab, cnt = _precompute_table(
        emb_table, hard_norm_scale.reshape(1, -1), proj_w,
        ids.reshape(n_ranges, -1), n_ranges, n_slices,
    )
    out = _make_broadcast_write(bsz * seq, emb_table.shape[0], f)(
        ids, ptab, cnt.reshape(-1)
    )
    return out.reshape(bsz, seq, f)


# speedup vs baseline: 1.3775x; 1.0002x over previous
import functools

import jax
import jax.numpy as jnp
from jax import lax
from jax.experimental import pallas as pl
from jax.experimental.pallas import tpu as pltpu
from jax.experimental.pallas import tpu_sc as plsc

_EPS = 1e-06

_NC = 2
_NS = 16
_NW = _NC * _NS

_N_SLICES = 8
_N_RANGES = _NW // _N_SLICES


def _precompute_body(n_ranges, n_slices, vs, emb_ref, scale_ref, w_ref,
                     ids_ref, out_ref, cnt_ref):
    x = emb_ref[...]
    var = jnp.mean(x * x, axis=-1, keepdims=True)
    y = x * lax.rsqrt(var + _EPS) * scale_ref[...]
    z = jnp.dot(y, w_ref[...], preferred_element_type=jnp.float32)
    var2 = jnp.mean(z * z, axis=-1, keepdims=True)
    out_ref[...] = z * lax.rsqrt(var2 + _EPS)
    ids = ids_ref[...]
    coli = lax.broadcasted_iota(jnp.int32, (1, 128), 1)
    acc = jnp.zeros((1, 128), jnp.int32)
    for r in range(n_ranges):
        row = ids[r]
        for s in range(n_slices):
            c = jnp.sum(((row >= s * vs) & (row < (s + 1) * vs))
                        .astype(jnp.int32))
            acc = acc + jnp.where(coli == r * n_slices + s, c, 0)
    cnt_ref[...] = acc


def _precompute_table(emb_table, scale2d, proj_w, ids2d):
    v, _ = emb_table.shape
    f = proj_w.shape[1]
    body = functools.partial(
        _precompute_body, _N_RANGES, _N_SLICES, v // _N_SLICES
    )
    return pl.pallas_call(
        body,
        out_shape=[
            jax.ShapeDtypeStruct((v, f), jnp.float32),
            jax.ShapeDtypeStruct((1, 128), jnp.int32),
        ],
    )(emb_table, scale2d, proj_w, ids2d)


@functools.lru_cache(maxsize=None)
def _make_broadcast_write(b, v, d):
    n_slices = _N_SLICES
    n_ranges = _N_RANGES
    vs = v // n_slices
    tr = b // n_ranges
    grp = tr // 16
    mesh = plsc.VectorSubcoreMesh(core_axis_name="c", subcore_axis_name="s")

    @functools.partial(
        pl.kernel,
        mesh=mesh,
        out_type=jax.ShapeDtypeStruct((b, d), jnp.float32),
        scratch_types=[
            pltpu.VMEM((tr,), jnp.int32),
            pltpu.VMEM((vs, d), jnp.float32),
            pltpu.VMEM((48,), jnp.int32),
            pltpu.SemaphoreType.DMA,
        ],
    )
    def bw_kernel(ids_hbm, tab_hbm, cnt_hbm, out_hbm, ids_v, tab_v, cnt_v,
                  sem):
        wid = lax.axis_index("s") * _NC + lax.axis_index("c")
        slice_id = wid % n_slices
        range_id = wid // n_slices
        v0 = slice_id * vs
        t0 = range_id * tr
        pltpu.sync_copy(tab_hbm.at[pl.ds(v0, vs)], tab_v)
        pltpu.sync_copy(ids_hbm.at[pl.ds(t0, tr)], ids_v)
        pltpu.sync_copy(cnt_hbm.at[pl.ds(0, 48)], cnt_v)
        n_issued = cnt_v[pl.ds(wid, 16)][0]

        def scan(g, z):
            ids16 = ids_v[pl.ds(g * 16, 16)]
            rows16 = ids16 - v0
            mask = (ids16 >= v0) & (ids16 < v0 + vs)
            enc16 = jnp.where(mask, rows16, -1)
            gbase = t0 + g * 16
            for q in range(0, 16, 4):
                e4 = [enc16[q + i] for i in range(4)]
                any_hit = e4[0] & e4[1] & e4[2] & e4[3]

                @pl.when(any_hit >= 0)
                def _(q=q, e4=e4):
                    for i in range(4):
                        e = e4[i]

                        @pl.when(e >= 0)
                        def _(e=e, l=q + i):
                            pltpu.async_copy(
                                tab_v.at[pl.ds(e, 1)],
                                out_hbm.at[pl.ds(gbase + l, 1)],
                                sem,
                            )

            return z

        lax.fori_loop(0, grp, scan, jnp.int32(0))

        def drain(_, carry):
            pltpu.make_async_copy(
                out_hbm.at[pl.ds(t0, 1)], tab_v.at[pl.ds(0, 1)], sem
            ).wait()
            return carry

        lax.fori_loop(0, n_issued, drain, jnp.int32(0))

    return bw_kernel


def kernel(input_ids, emb_table, hard_norm_scale, proj_w):
    bsz, seq = input_ids.shape
    f = proj_w.shape[1]
    ids = input_ids.reshape(-1).astype(jnp.int32)
    ptab, cnt = _precompute_table(
        emb_table, hard_norm_scale.reshape(1, -1), proj_w,
        ids.reshape(_N_RANGES, -1),
    )
    out = _make_broadcast_write(bsz * seq, emb_table.shape[0], f)(
        ids, ptab, cnt.reshape(-1)
    )
    return out.reshape(bsz, seq, f)
